# baseline (speedup 1.0000x reference)
import functools
import math

import jax
import jax.numpy as jnp
from jax import lax
from jax.experimental import pallas as pl
from jax.experimental.pallas import tpu as pltpu
from jax.experimental.pallas import tpu_sc as plsc

KNN = 20
EPSB = 1e-5
NEG_BIG = -3.0e38
NUM_WORKERS = 32
CP = 128


def _knn_tc(xt, xx, tile=256, interpret=False):
    B, N, _ = xt.shape
    nt = N // tile

    def body(xfull_ref, xtile_ref, xx_ref, xxt_ref, idx_ref):
        b = pl.program_id(0)
        xfull = xfull_ref[0]
        xtile = xtile_ref[0]
        xx_full = xx_ref[0, 0]
        xx_tile = xxt_ref[0, 0]
        dot = lax.dot_general(xtile, xfull, (((1,), (1,)), ((), ())),
                              preferred_element_type=jnp.float32)
        pd = 2.0 * dot - xx_tile[:, None] - xx_full[None, :]
        colf = lax.broadcasted_iota(jnp.int32, (tile, N), 1).astype(jnp.float32)
        big = jnp.float32(3.0e38)
        cols = []
        m = jnp.max(pd, axis=1, keepdims=True)
        for s in range(KNN):
            sel = jnp.min(jnp.where(pd == m, colf, big), axis=1, keepdims=True)
            cols.append(sel)
            if s + 1 < KNN:
                pd = jnp.where(colf == sel, NEG_BIG, pd)
                m = jnp.max(pd, axis=1, keepdims=True)
        idxf = jnp.concatenate(cols, axis=1)
        idx_ref[0] = idxf.astype(jnp.int32) + b * N

    return pl.pallas_call(
        body,
        grid=(B, nt),
        in_specs=[
            pl.BlockSpec((1, N, CP), lambda b, t: (b, 0, 0)),
            pl.BlockSpec((1, tile, CP), lambda b, t: (b, t, 0)),
            pl.BlockSpec((1, 1, N), lambda b, t: (b, 0, 0)),
            pl.BlockSpec((1, 1, tile), lambda b, t: (b, 0, t)),
        ],
        out_specs=pl.BlockSpec((1, tile, KNN), lambda b, t: (b, t, 0)),
        out_shape=jax.ShapeDtypeStruct((B, N, KNN), jnp.int32),
        interpret=interpret,
    )(xt, xt, xx, xx)


def _gather_sc(table, idx, chunk=16):
    M, _ = table.shape
    per_w = M // NUM_WORKERS
    nch = per_w // chunk
    mesh = plsc.VectorSubcoreMesh(core_axis_name="c", subcore_axis_name="s")

    @functools.partial(
        pl.kernel,
        mesh=mesh,
        out_type=jax.ShapeDtypeStruct((M, KNN, CP), jnp.float32),
        scratch_types=[
            pltpu.VMEM((chunk, KNN), jnp.int32),
            pltpu.VMEM((chunk, KNN, CP), jnp.float32),
            pltpu.SemaphoreType.DMA,
        ],
    )
    def run(tab_hbm, idx_hbm, out_hbm, ibuf, rbuf, sem):
        wid = lax.axis_index("s") * 2 + lax.axis_index("c")
        base0 = wid * per_w

        def do_chunk(ch, carry):
            base = base0 + ch * chunk
            pltpu.sync_copy(idx_hbm.at[pl.ds(base, chunk)], ibuf)
            for p in range(chunk):
                pltpu.async_copy(tab_hbm.at[ibuf.at[p]], rbuf.at[p], sem)
            for p in range(chunk):
                pltpu.make_async_copy(tab_hbm.at[ibuf.at[p]], rbuf.at[p], sem).wait()
            pltpu.sync_copy(rbuf, out_hbm.at[pl.ds(base, chunk)])
            return carry

        lax.fori_loop(0, nch, do_chunk, 0)

    return run(table, idx)


def _edgeconv_tc(gat, xt, w_t, g, b, tile=256, interpret=False):
    B, N, _, _ = gat.shape
    nt = N // tile
    kh = KNN // 2
    gflat = gat.reshape(B, N, KNN * CP)

    def body(g0_ref, g1_ref, x_ref, w_ref, gg_ref, bb_ref, o_ref, xx_ref):
        sqrtc = jnp.sqrt(jnp.float32(1.0 + EPSB))
        xj = jnp.concatenate([g0_ref[0].reshape(tile, kh, CP),
                              g1_ref[0].reshape(tile, kh, CP)], axis=1)
        xi = x_ref[0]
        xib = jnp.broadcast_to(xi[:, None, :], xj.shape)
        f = jnp.concatenate([xj - xib, xib], axis=-1)
        ff = f.reshape(tile * KNN, 2 * CP)
        y = lax.dot_general(ff, w_ref[...], (((1,), (0,)), ((), ())),
                            preferred_element_type=jnp.float32)
        y = gg_ref[...][None, :] * (y / sqrtc) + bb_ref[...][None, :]
        y = jnp.where(y > 0.0, y, 0.2 * y)
        xn = jnp.max(y.reshape(tile, KNN, CP), axis=1)
        o_ref[0] = xn
        xx_ref[0, 0] = jnp.sum(xn * xn, axis=1)

    return pl.pallas_call(
        body,
        grid=(B, nt),
        in_specs=[
            pl.BlockSpec((1, tile, kh * CP), lambda b, t: (b, t, 0)),
            pl.BlockSpec((1, tile, kh * CP), lambda b, t: (b, t, 1)),
            pl.BlockSpec((1, tile, CP), lambda b, t: (b, t, 0)),
            pl.BlockSpec((2 * CP, CP), lambda b, t: (0, 0)),
            pl.BlockSpec((CP,), lambda b, t: (0,)),
            pl.BlockSpec((CP,), lambda b, t: (0,)),
        ],
        out_specs=[
            pl.BlockSpec((1, tile, CP), lambda b, t: (b, t, 0)),
            pl.BlockSpec((1, 1, tile), lambda b, t: (b, 0, t)),
        ],
        out_shape=[
            jax.ShapeDtypeStruct((B, N, CP), jnp.float32),
            jax.ShapeDtypeStruct((B, 1, N), jnp.float32),
        ],
        interpret=interpret,
    )(gflat, gflat, xt, w_t, g, b)


def _head_tc(xc, w5_t, g5, b5, l1_t, g6, b6, l2_t, l2b, g7, b7, l3_t, l3b,
             interpret=False):
    B, N, C5 = xc.shape
    H5 = w5_t.shape[1]
    ncls = l3_t.shape[1]

    def body(xc_ref, w5_ref, g5_ref, b5_ref, l1_ref, g6_ref, b6_ref,
             l2_ref, l2b_ref, g7_ref, b7_ref, l3_ref, l3b_ref, out_ref):
        sqrtc = jnp.sqrt(jnp.float32(1.0 + EPSB))
        x = xc_ref[0]
        y = lax.dot_general(x, w5_ref[...], (((1,), (0,)), ((), ())),
                            preferred_element_type=jnp.float32)
        y = g5_ref[...][None, :] * (y / sqrtc) + b5_ref[...][None, :]
        y = jnp.where(y > 0.0, y, 0.2 * y)
        p1 = jnp.max(y, axis=0)
        p2 = jnp.sum(y, axis=0) / jnp.float32(N)
        z = jnp.concatenate([p1, p2])[None, :]
        h = lax.dot_general(z, l1_ref[...], (((1,), (0,)), ((), ())),
                            preferred_element_type=jnp.float32)
        h = g6_ref[...][None, :] * (h / sqrtc) + b6_ref[...][None, :]
        h = jnp.where(h > 0.0, h, 0.2 * h)
        h = lax.dot_general(h, l2_ref[...], (((1,), (0,)), ((), ())),
                            preferred_element_type=jnp.float32) + l2b_ref[...][None, :]
        h = g7_ref[...][None, :] * (h / sqrtc) + b7_ref[...][None, :]
        h = jnp.where(h > 0.0, h, 0.2 * h)
        o = lax.dot_general(h, l3_ref[...], (((1,), (0,)), ((), ())),
                            preferred_element_type=jnp.float32) + l3b_ref[...][None, :]
        out_ref[0] = o

    whole = lambda *s: pl.BlockSpec(s, lambda bb: tuple(0 for _ in s))
    return pl.pallas_call(
        body,
        grid=(B,),
        in_specs=[
            pl.BlockSpec((1, N, C5), lambda bb: (bb, 0, 0)),
            whole(C5, H5), whole(H5), whole(H5),
            whole(2 * H5, 512), whole(512), whole(512),
            whole(512, 256), whole(256), whole(256), whole(256),
            whole(256, ncls), whole(ncls),
        ],
        out_specs=pl.BlockSpec((1, 1, ncls), lambda bb: (bb, 0, 0)),
        out_shape=jax.ShapeDtypeStruct((B, 1, ncls), jnp.float32),
        interpret=interpret,
    )(xc, w5_t, g5, b5, l1_t, g6, b6, l2_t, l2b, g7, b7, l3_t, l3b)[:, 0, :]


def _pad_stage_weights(W, cin):
    co = W.shape[0]
    wt = jnp.zeros((2 * CP, CP), jnp.float32)
    wt = wt.at[:cin, :co].set(W[:, :cin].T)
    wt = wt.at[CP:CP + cin, :co].set(W[:, cin:].T)
    return wt


def _edge_stage_grouped(parts, W, g, b, cin):
    co = W.shape[0]
    wt = _pad_stage_weights(W, cin)
    gp = jnp.pad(g, (0, CP - co))
    bp = jnp.pad(b, (0, CP - co))
    out = []
    for xt, xx in parts:
        GS, N, _ = xt.shape
        idx = _knn_tc(xt, xx)
        gat = _gather_sc(xt.reshape(GS * N, CP), idx.reshape(GS * N, KNN))
        out.append(_edgeconv_tc(gat.reshape(GS, N, KNN, CP), xt, wt, gp, bp))
    return out


def kernel(xyz, W1, g1, b1, W2, g2, b2, W3, g3, b3, W4, g4, b4, W5, g5, b5,
           L1w, g6, b6, L2w, L2b, g7, b7, L3w, L3b):
    B, N, _ = xyz.shape
    GS = 2
    x0 = jnp.pad(xyz, ((0, 0), (0, 0), (0, CP - 3)))
    xx0 = jnp.sum(x0 * x0, axis=2)[:, None, :]
    p0 = [(x0[lo:lo + GS], xx0[lo:lo + GS]) for lo in range(0, B, GS)]
    p1 = _edge_stage_grouped(p0, W1, g1, b1, cin=3)
    p2 = _edge_stage_grouped(p1, W2, g2, b2, cin=64)
    p3 = _edge_stage_grouped(p2, W3, g3, b3, cin=64)
    p4 = _edge_stage_grouped(p3, W4, g4, b4, cin=64)
    x1 = jnp.concatenate([p[0] for p in p1], axis=0)
    x2 = jnp.concatenate([p[0] for p in p2], axis=0)
    x3 = jnp.concatenate([p[0] for p in p3], axis=0)
    x4 = jnp.concatenate([p[0] for p in p4], axis=0)
    xc = jnp.concatenate([x1[..., :64], x2[..., :64], x3[..., :64], x4],
                         axis=2)
    z = _head_tc(xc, W5.T, g5, b5, L1w.T, g6, b6, L2w.T, L2b, g7, b7,
                 L3w.T, L3b)
    return jnp.broadcast_to(z[:, None, :], (B, N, z.shape[-1]))


# speedup vs baseline: 1.2938x; 1.2938x over previous
import functools
import math

import jax
import jax.numpy as jnp
from jax import lax
from jax.experimental import pallas as pl
from jax.experimental.pallas import tpu as pltpu
from jax.experimental.pallas import tpu_sc as plsc

KNN = 20
EPSB = 1e-5
NEG_BIG = -3.0e38
NUM_WORKERS = 32
CP = 128


def _knn_tc(xt, xx, tile=256, interpret=False):
    B, N, _ = xt.shape
    nt = N // tile

    def body(xfull_ref, xtile_ref, xx_ref, xxt_ref, idx_ref):
        b = pl.program_id(0)
        xfull = xfull_ref[0]
        xtile = xtile_ref[0]
        xx_full = xx_ref[0, 0]
        xx_tile = xxt_ref[0, 0]
        dot = lax.dot_general(xtile, xfull, (((1,), (1,)), ((), ())),
                              preferred_element_type=jnp.float32)
        pd = 2.0 * dot - xx_tile[:, None] - xx_full[None, :]
        colf = lax.broadcasted_iota(jnp.int32, (tile, N), 1).astype(jnp.float32)
        big = jnp.float32(3.0e38)
        cols = []
        m = jnp.max(pd, axis=1, keepdims=True)
        for s in range(KNN):
            sel = jnp.min(jnp.where(pd == m, colf, big), axis=1, keepdims=True)
            cols.append(sel)
            if s + 1 < KNN:
                pd = jnp.where(colf == sel, NEG_BIG, pd)
                m = jnp.max(pd, axis=1, keepdims=True)
        idxf = jnp.concatenate(cols, axis=1)
        idx_ref[0] = idxf.astype(jnp.int32) + b * N

    return pl.pallas_call(
        body,
        grid=(B, nt),
        in_specs=[
            pl.BlockSpec((1, N, CP), lambda b, t: (b, 0, 0)),
            pl.BlockSpec((1, tile, CP), lambda b, t: (b, t, 0)),
            pl.BlockSpec((1, 1, N), lambda b, t: (b, 0, 0)),
            pl.BlockSpec((1, 1, tile), lambda b, t: (b, 0, t)),
        ],
        out_specs=pl.BlockSpec((1, tile, KNN), lambda b, t: (b, t, 0)),
        out_shape=jax.ShapeDtypeStruct((B, N, KNN), jnp.int32),
        interpret=interpret,
    )(xt, xt, xx, xx)


def _gather_sc(table, idx, chunk=16):
    M, _ = table.shape
    per_w = M // NUM_WORKERS
    nch = per_w // chunk
    mesh = plsc.VectorSubcoreMesh(core_axis_name="c", subcore_axis_name="s")

    @functools.partial(
        pl.kernel,
        mesh=mesh,
        out_type=jax.ShapeDtypeStruct((M, KNN, CP), jnp.float32),
        scratch_types=[
            pltpu.VMEM((chunk, KNN), jnp.int32),
            pltpu.VMEM((chunk, KNN, CP), jnp.float32),
            pltpu.SemaphoreType.DMA,
        ],
    )
    def run(tab_hbm, idx_hbm, out_hbm, ibuf, rbuf, sem):
        wid = lax.axis_index("s") * 2 + lax.axis_index("c")
        base0 = wid * per_w

        def do_chunk(ch, carry):
            base = base0 + ch * chunk
            pltpu.sync_copy(idx_hbm.at[pl.ds(base, chunk)], ibuf)
            for p in range(chunk):
                pltpu.async_copy(tab_hbm.at[ibuf.at[p]], rbuf.at[p], sem)
            for p in range(chunk):
                pltpu.make_async_copy(tab_hbm.at[ibuf.at[p]], rbuf.at[p], sem).wait()
            pltpu.sync_copy(rbuf, out_hbm.at[pl.ds(base, chunk)])
            return carry

        lax.fori_loop(0, nch, do_chunk, 0)

    return run(table, idx)


def _edgeconv_tc(gat, xt, w_t, g, b, tile=256, interpret=False):
    B, N, _, _ = gat.shape
    nt = N // tile

    def body(g_ref, x_ref, w_ref, gg_ref, bb_ref, o_ref, xx_ref):
        sqrtc = jnp.sqrt(jnp.float32(1.0 + EPSB))
        xj = g_ref[0]
        xi = x_ref[0]
        xib = jnp.broadcast_to(xi[:, None, :], xj.shape)
        f = jnp.concatenate([xj - xib, xib], axis=-1)
        ff = f.reshape(tile * KNN, 2 * CP)
        y = lax.dot_general(ff, w_ref[...], (((1,), (0,)), ((), ())),
                            preferred_element_type=jnp.float32)
        y = gg_ref[...][None, :] * (y / sqrtc) + bb_ref[...][None, :]
        y = jnp.where(y > 0.0, y, 0.2 * y)
        xn = jnp.max(y.reshape(tile, KNN, CP), axis=1)
        o_ref[0] = xn
        xx_ref[0, 0] = jnp.sum(xn * xn, axis=1)

    return pl.pallas_call(
        body,
        grid=(B, nt),
        in_specs=[
            pl.BlockSpec((1, tile, KNN, CP), lambda b, t: (b, t, 0, 0)),
            pl.BlockSpec((1, tile, CP), lambda b, t: (b, t, 0)),
            pl.BlockSpec((2 * CP, CP), lambda b, t: (0, 0)),
            pl.BlockSpec((CP,), lambda b, t: (0,)),
            pl.BlockSpec((CP,), lambda b, t: (0,)),
        ],
        out_specs=[
            pl.BlockSpec((1, tile, CP), lambda b, t: (b, t, 0)),
            pl.BlockSpec((1, 1, tile), lambda b, t: (b, 0, t)),
        ],
        out_shape=[
            jax.ShapeDtypeStruct((B, N, CP), jnp.float32),
            jax.ShapeDtypeStruct((B, 1, N), jnp.float32),
        ],
        interpret=interpret,
    )(gat, xt, w_t, g, b)


def _head_tc(xc, w5_t, g5, b5, l1_t, g6, b6, l2_t, l2b, g7, b7, l3_t, l3b,
             interpret=False):
    B, N, C5 = xc.shape
    H5 = w5_t.shape[1]
    ncls = l3_t.shape[1]

    def body(xc_ref, w5_ref, g5_ref, b5_ref, l1_ref, g6_ref, b6_ref,
             l2_ref, l2b_ref, g7_ref, b7_ref, l3_ref, l3b_ref, out_ref):
        sqrtc = jnp.sqrt(jnp.float32(1.0 + EPSB))
        x = xc_ref[0]
        y = lax.dot_general(x, w5_ref[...], (((1,), (0,)), ((), ())),
                            preferred_element_type=jnp.float32)
        y = g5_ref[...][None, :] * (y / sqrtc) + b5_ref[...][None, :]
        y = jnp.where(y > 0.0, y, 0.2 * y)
        p1 = jnp.max(y, axis=0)
        p2 = jnp.sum(y, axis=0) / jnp.float32(N)
        z = jnp.concatenate([p1, p2])[None, :]
        h = lax.dot_general(z, l1_ref[...], (((1,), (0,)), ((), ())),
                            preferred_element_type=jnp.float32)
        h = g6_ref[...][None, :] * (h / sqrtc) + b6_ref[...][None, :]
        h = jnp.where(h > 0.0, h, 0.2 * h)
        h = lax.dot_general(h, l2_ref[...], (((1,), (0,)), ((), ())),
                            preferred_element_type=jnp.float32) + l2b_ref[...][None, :]
        h = g7_ref[...][None, :] * (h / sqrtc) + b7_ref[...][None, :]
        h = jnp.where(h > 0.0, h, 0.2 * h)
        o = lax.dot_general(h, l3_ref[...], (((1,), (0,)), ((), ())),
                            preferred_element_type=jnp.float32) + l3b_ref[...][None, :]
        out_ref[0] = o

    whole = lambda *s: pl.BlockSpec(s, lambda bb: tuple(0 for _ in s))
    return pl.pallas_call(
        body,
        grid=(B,),
        in_specs=[
            pl.BlockSpec((1, N, C5), lambda bb: (bb, 0, 0)),
            whole(C5, H5), whole(H5), whole(H5),
            whole(2 * H5, 512), whole(512), whole(512),
            whole(512, 256), whole(256), whole(256), whole(256),
            whole(256, ncls), whole(ncls),
        ],
        out_specs=pl.BlockSpec((1, 1, ncls), lambda bb: (bb, 0, 0)),
        out_shape=jax.ShapeDtypeStruct((B, 1, ncls), jnp.float32),
        interpret=interpret,
    )(xc, w5_t, g5, b5, l1_t, g6, b6, l2_t, l2b, g7, b7, l3_t, l3b)[:, 0, :]


def _pad_stage_weights(W, cin):
    co = W.shape[0]
    wt = jnp.zeros((2 * CP, CP), jnp.float32)
    wt = wt.at[:cin, :co].set(W[:, :cin].T)
    wt = wt.at[CP:CP + cin, :co].set(W[:, cin:].T)
    return wt


def _edge_stage_grouped(parts, W, g, b, cin):
    co = W.shape[0]
    wt = _pad_stage_weights(W, cin)
    gp = jnp.pad(g, (0, CP - co))
    bp = jnp.pad(b, (0, CP - co))
    out = []
    for xt, xx in parts:
        GS, N, _ = xt.shape
        idx = _knn_tc(xt, xx)
        gat = _gather_sc(xt.reshape(GS * N, CP), idx.reshape(GS * N, KNN))
        out.append(_edgeconv_tc(gat.reshape(GS, N, KNN, CP), xt, wt, gp, bp))
    return out


def kernel(xyz, W1, g1, b1, W2, g2, b2, W3, g3, b3, W4, g4, b4, W5, g5, b5,
           L1w, g6, b6, L2w, L2b, g7, b7, L3w, L3b):
    B, N, _ = xyz.shape
    GS = 2
    x0 = jnp.pad(xyz, ((0, 0), (0, 0), (0, CP - 3)))
    xx0 = jnp.sum(x0 * x0, axis=2)[:, None, :]
    p0 = [(x0[lo:lo + GS], xx0[lo:lo + GS]) for lo in range(0, B, GS)]
    p1 = _edge_stage_grouped(p0, W1, g1, b1, cin=3)
    p2 = _edge_stage_grouped(p1, W2, g2, b2, cin=64)
    p3 = _edge_stage_grouped(p2, W3, g3, b3, cin=64)
    p4 = _edge_stage_grouped(p3, W4, g4, b4, cin=64)
    x1 = jnp.concatenate([p[0] for p in p1], axis=0)
    x2 = jnp.concatenate([p[0] for p in p2], axis=0)
    x3 = jnp.concatenate([p[0] for p in p3], axis=0)
    x4 = jnp.concatenate([p[0] for p in p4], axis=0)
    xc = jnp.concatenate([x1[..., :64], x2[..., :64], x3[..., :64], x4],
                         axis=2)
    z = _head_tc(xc, W5.T, g5, b5, L1w.T, g6, b6, L2w.T, L2b, g7, b7,
                 L3w.T, L3b)
    return jnp.broadcast_to(z[:, None, :], (B, N, z.shape[-1]))


# speedup vs baseline: 1.3476x; 1.0416x over previous
import functools
import math

import jax
import jax.numpy as jnp
from jax import lax
from jax.experimental import pallas as pl
from jax.experimental.pallas import tpu as pltpu
from jax.experimental.pallas import tpu_sc as plsc

KNN = 20
EPSB = 1e-5
NEG_BIG = -3.0e38
NUM_WORKERS = 32
CP = 128


def _knn_tc(xt, xx, tile=512, interpret=False):
    B, N, _ = xt.shape
    nt = N // tile

    def body(xfull_ref, xtile_ref, xx_ref, xxt_ref, idx_ref):
        b = pl.program_id(0)
        xfull = xfull_ref[0]
        xtile = xtile_ref[0]
        xx_full = xx_ref[0, 0]
        xx_tile = xxt_ref[0, 0]
        dot = lax.dot_general(xtile, xfull, (((1,), (1,)), ((), ())),
                              preferred_element_type=jnp.float32)
        pd = 2.0 * dot - xx_tile[:, None] - xx_full[None, :]
        colf = lax.broadcasted_iota(jnp.int32, (tile, N), 1).astype(jnp.float32)
        big = jnp.float32(3.0e38)
        cols = []
        m = jnp.max(pd, axis=1, keepdims=True)
        for s in range(KNN):
            sel = jnp.min(jnp.where(pd == m, colf, big), axis=1, keepdims=True)
            cols.append(sel)
            if s + 1 < KNN:
                pd = jnp.where(colf == sel, NEG_BIG, pd)
                m = jnp.max(pd, axis=1, keepdims=True)
        idxf = jnp.concatenate(cols, axis=1)
        idx_ref[0] = idxf.astype(jnp.int32) + b * N

    return pl.pallas_call(
        body,
        grid=(B, nt),
        in_specs=[
            pl.BlockSpec((1, N, CP), lambda b, t: (b, 0, 0)),
            pl.BlockSpec((1, tile, CP), lambda b, t: (b, t, 0)),
            pl.BlockSpec((1, 1, N), lambda b, t: (b, 0, 0)),
            pl.BlockSpec((1, 1, tile), lambda b, t: (b, 0, t)),
        ],
        out_specs=pl.BlockSpec((1, tile, KNN), lambda b, t: (b, t, 0)),
        out_shape=jax.ShapeDtypeStruct((B, N, KNN), jnp.int32),
        interpret=interpret,
    )(xt, xt, xx, xx)


def _gather_sc(table, idx, chunk=16):
    M, _ = table.shape
    per_w = M // NUM_WORKERS
    nch = per_w // chunk
    mesh = plsc.VectorSubcoreMesh(core_axis_name="c", subcore_axis_name="s")

    @functools.partial(
        pl.kernel,
        mesh=mesh,
        out_type=jax.ShapeDtypeStruct((M, KNN, CP), jnp.float32),
        scratch_types=[
            pltpu.VMEM((chunk, KNN), jnp.int32),
            pltpu.VMEM((chunk, KNN, CP), jnp.float32),
            pltpu.SemaphoreType.DMA,
        ],
    )
    def run(tab_hbm, idx_hbm, out_hbm, ibuf, rbuf, sem):
        wid = lax.axis_index("s") * 2 + lax.axis_index("c")
        base0 = wid * per_w

        def do_chunk(ch, carry):
            base = base0 + ch * chunk
            pltpu.sync_copy(idx_hbm.at[pl.ds(base, chunk)], ibuf)
            for p in range(chunk):
                pltpu.async_copy(tab_hbm.at[ibuf.at[p]], rbuf.at[p], sem)
            for p in range(chunk):
                pltpu.make_async_copy(tab_hbm.at[ibuf.at[p]], rbuf.at[p], sem).wait()
            pltpu.sync_copy(rbuf, out_hbm.at[pl.ds(base, chunk)])
            return carry

        lax.fori_loop(0, nch, do_chunk, 0)

    return run(table, idx)


def _edgeconv_tc(gat, xt, w_t, g, b, tile=512, interpret=False):
    B, N, _, _ = gat.shape
    nt = N // tile

    def body(g_ref, x_ref, w_ref, gg_ref, bb_ref, o_ref, xx_ref):
        sqrtc = jnp.sqrt(jnp.float32(1.0 + EPSB))
        xj = g_ref[0]
        xi = x_ref[0]
        xib = jnp.broadcast_to(xi[:, None, :], xj.shape)
        f = jnp.concatenate([xj - xib, xib], axis=-1)
        ff = f.reshape(tile * KNN, 2 * CP)
        y = lax.dot_general(ff, w_ref[...], (((1,), (0,)), ((), ())),
                            preferred_element_type=jnp.float32)
        y = gg_ref[...][None, :] * (y / sqrtc) + bb_ref[...][None, :]
        y = jnp.where(y > 0.0, y, 0.2 * y)
        xn = jnp.max(y.reshape(tile, KNN, CP), axis=1)
        o_ref[0] = xn
        xx_ref[0, 0] = jnp.sum(xn * xn, axis=1)

    return pl.pallas_call(
        body,
        grid=(B, nt),
        in_specs=[
            pl.BlockSpec((1, tile, KNN, CP), lambda b, t: (b, t, 0, 0)),
            pl.BlockSpec((1, tile, CP), lambda b, t: (b, t, 0)),
            pl.BlockSpec((2 * CP, CP), lambda b, t: (0, 0)),
            pl.BlockSpec((CP,), lambda b, t: (0,)),
            pl.BlockSpec((CP,), lambda b, t: (0,)),
        ],
        out_specs=[
            pl.BlockSpec((1, tile, CP), lambda b, t: (b, t, 0)),
            pl.BlockSpec((1, 1, tile), lambda b, t: (b, 0, t)),
        ],
        out_shape=[
            jax.ShapeDtypeStruct((B, N, CP), jnp.float32),
            jax.ShapeDtypeStruct((B, 1, N), jnp.float32),
        ],
        interpret=interpret,
    )(gat, xt, w_t, g, b)


def _head_tc(xc, w5_t, g5, b5, l1_t, g6, b6, l2_t, l2b, g7, b7, l3_t, l3b,
             interpret=False):
    B, N, C5 = xc.shape
    H5 = w5_t.shape[1]
    ncls = l3_t.shape[1]

    def body(xc_ref, w5_ref, g5_ref, b5_ref, l1_ref, g6_ref, b6_ref,
             l2_ref, l2b_ref, g7_ref, b7_ref, l3_ref, l3b_ref, out_ref):
        sqrtc = jnp.sqrt(jnp.float32(1.0 + EPSB))
        x = xc_ref[0]
        y = lax.dot_general(x, w5_ref[...], (((1,), (0,)), ((), ())),
                            preferred_element_type=jnp.float32)
        y = g5_ref[...][None, :] * (y / sqrtc) + b5_ref[...][None, :]
        y = jnp.where(y > 0.0, y, 0.2 * y)
        p1 = jnp.max(y, axis=0)
        p2 = jnp.sum(y, axis=0) / jnp.float32(N)
        z = jnp.concatenate([p1, p2])[None, :]
        h = lax.dot_general(z, l1_ref[...], (((1,), (0,)), ((), ())),
                            preferred_element_type=jnp.float32)
        h = g6_ref[...][None, :] * (h / sqrtc) + b6_ref[...][None, :]
        h = jnp.where(h > 0.0, h, 0.2 * h)
        h = lax.dot_general(h, l2_ref[...], (((1,), (0,)), ((), ())),
                            preferred_element_type=jnp.float32) + l2b_ref[...][None, :]
        h = g7_ref[...][None, :] * (h / sqrtc) + b7_ref[...][None, :]
        h = jnp.where(h > 0.0, h, 0.2 * h)
        o = lax.dot_general(h, l3_ref[...], (((1,), (0,)), ((), ())),
                            preferred_element_type=jnp.float32) + l3b_ref[...][None, :]
        out_ref[0] = o

    whole = lambda *s: pl.BlockSpec(s, lambda bb: tuple(0 for _ in s))
    return pl.pallas_call(
        body,
        grid=(B,),
        in_specs=[
            pl.BlockSpec((1, N, C5), lambda bb: (bb, 0, 0)),
            whole(C5, H5), whole(H5), whole(H5),
            whole(2 * H5, 512), whole(512), whole(512),
            whole(512, 256), whole(256), whole(256), whole(256),
            whole(256, ncls), whole(ncls),
        ],
        out_specs=pl.BlockSpec((1, 1, ncls), lambda bb: (bb, 0, 0)),
        out_shape=jax.ShapeDtypeStruct((B, 1, ncls), jnp.float32),
        interpret=interpret,
    )(xc, w5_t, g5, b5, l1_t, g6, b6, l2_t, l2b, g7, b7, l3_t, l3b)[:, 0, :]


def _pad_stage_weights(W, cin):
    co = W.shape[0]
    wt = jnp.zeros((2 * CP, CP), jnp.float32)
    wt = wt.at[:cin, :co].set(W[:, :cin].T)
    wt = wt.at[CP:CP + cin, :co].set(W[:, cin:].T)
    return wt


def _edge_stage_grouped(parts, W, g, b, cin):
    co = W.shape[0]
    wt = _pad_stage_weights(W, cin)
    gp = jnp.pad(g, (0, CP - co))
    bp = jnp.pad(b, (0, CP - co))
    out = []
    for xt, xx in parts:
        GS, N, _ = xt.shape
        idx = _knn_tc(xt, xx)
        gat = _gather_sc(xt.reshape(GS * N, CP), idx.reshape(GS * N, KNN))
        out.append(_edgeconv_tc(gat.reshape(GS, N, KNN, CP), xt, wt, gp, bp))
    return out


def kernel(xyz, W1, g1, b1, W2, g2, b2, W3, g3, b3, W4, g4, b4, W5, g5, b5,
           L1w, g6, b6, L2w, L2b, g7, b7, L3w, L3b):
    B, N, _ = xyz.shape
    GS = 2
    x0 = jnp.pad(xyz, ((0, 0), (0, 0), (0, CP - 3)))
    xx0 = jnp.sum(x0 * x0, axis=2)[:, None, :]
    p0 = [(x0[lo:lo + GS], xx0[lo:lo + GS]) for lo in range(0, B, GS)]
    p1 = _edge_stage_grouped(p0, W1, g1, b1, cin=3)
    p2 = _edge_stage_grouped(p1, W2, g2, b2, cin=64)
    p3 = _edge_stage_grouped(p2, W3, g3, b3, cin=64)
    p4 = _edge_stage_grouped(p3, W4, g4, b4, cin=64)
    x1 = jnp.concatenate([p[0] for p in p1], axis=0)
    x2 = jnp.concatenate([p[0] for p in p2], axis=0)
    x3 = jnp.concatenate([p[0] for p in p3], axis=0)
    x4 = jnp.concatenate([p[0] for p in p4], axis=0)
    xc = jnp.concatenate([x1[..., :64], x2[..., :64], x3[..., :64], x4],
                         axis=2)
    z = _head_tc(xc, W5.T, g5, b5, L1w.T, g6, b6, L2w.T, L2b, g7, b7,
                 L3w.T, L3b)
    return jnp.broadcast_to(z[:, None, :], (B, N, z.shape[-1]))
